# baseline (speedup 1.0000x reference)
import functools

import jax
import jax.numpy as jnp
from jax import lax
from jax.experimental import pallas as pl
from jax.experimental.pallas import tpu as pltpu
from jax.experimental.pallas import tpu_sc as plsc

_K = 8
_P = 2
_G = 16
_S = 125
_BN = _P * _G * _S
_NW = 32
_LANES = 16


def _topk_body(mask_ref, x_ref, k_ref, w_ref, i_ref, xn_ref, rv_ref, ri_ref):
    q, _ = xn_ref.shape
    blk = pl.program_id(0)
    nblk = pl.num_programs(0)

    @pl.when(blk == 0)
    def _init():
        xx = x_ref[...]
        nrm = jnp.sqrt(jnp.sum(xx * xx, axis=1, keepdims=True))
        xn_ref[...] = xx / jnp.maximum(nrm, 1e-12)
        rv_ref[...] = jnp.full(rv_ref.shape, -jnp.inf, jnp.float32)
        ri_ref[...] = jnp.zeros(ri_ref.shape, jnp.float32)

    xn = xn_ref[...]
    cols = lax.broadcasted_iota(jnp.int32, (q, _S), 1).astype(jnp.float32)
    sss, pms, segs = [], [], []
    for p in range(_P):
        ssp = []
        for i in range(_G):
            kb = k_ref[p * _G + i]
            kn = jnp.sqrt(jnp.sum(kb * kb, axis=1, keepdims=True))
            kbn = kb / jnp.maximum(kn, 1e-12)
            ssp.append(lax.dot_general(xn, kbn, (((1,), (1,)), ((), ())),
                                       preferred_element_type=jnp.float32))
        sss.append(ssp)
        pm = ssp[0]
        for st in ssp[1:]:
            pm = jnp.maximum(pm, st)
        pms.append(pm)
        segs.append([])

    for _ in range(_K):
        for p in range(_P):
            m = jnp.max(pms[p], axis=1, keepdims=True)
            eq = pms[p] == m
            j = jnp.min(jnp.where(eq, cols, jnp.inf), axis=1, keepdims=True)
            pms[p] = jnp.where(eq, -jnp.inf, pms[p])
            segs[p].append(j)
    segfs = [jnp.concatenate(s, axis=1) for s in segs]

    base = blk * _BN
    cvals, reps = [], []
    for p in range(_P):
        sidx = segfs[p].astype(jnp.int32)
        cvals += [jnp.take_along_axis(s, sidx, axis=1) for s in sss[p]]
        reps += [segfs[p]] * _G
    pool = jnp.concatenate(cvals + [rv_ref[...]], axis=1)
    npool = _P * _G * _K + _K
    lane = lax.broadcasted_iota(jnp.int32, (q, npool), 1)
    offs = jnp.where(lane < _P * _G * _K, (lane // _K) * _S + base, 0)
    segrep = jnp.concatenate(reps + [ri_ref[...]], axis=1)
    poolidx = segrep + offs.astype(jnp.float32)
    ms, gs = [], []
    for _ in range(_K):
        m = jnp.max(pool, axis=1, keepdims=True)
        eq = pool == m
        g = jnp.min(jnp.where(eq, poolidx, jnp.inf), axis=1, keepdims=True)
        pool = jnp.where(eq, -jnp.inf, pool)
        ms.append(m)
        gs.append(g)
    rv_ref[...] = jnp.concatenate(ms, axis=1)
    ri_ref[...] = jnp.concatenate(gs, axis=1)

    @pl.when(blk == nblk - 1)
    def _fin():
        rv = rv_ref[...]
        valid = mask_ref[...] > 0.0
        mv = jnp.where(valid, rv, -jnp.inf)
        e = jnp.exp(mv - mv[:, :1])
        w_ref[...] = e / jnp.sum(e, axis=1, keepdims=True)
        i_ref[...] = ri_ref[...].astype(jnp.int32)


def _topk_call(mask, x, keys):
    q, d = x.shape
    n = keys.shape[0]
    nblk = n // _BN
    k3 = keys.reshape(nblk * _P * _G, _S, d)
    return pl.pallas_call(
        _topk_body,
        grid=(nblk,),
        in_specs=[
            pl.BlockSpec((1, _K), lambda i: (0, 0)),
            pl.BlockSpec((q, d), lambda i: (0, 0)),
            pl.BlockSpec((_P * _G, _S, d), lambda i: (i, 0, 0)),
        ],
        out_specs=[
            pl.BlockSpec((q, _K), lambda i: (0, 0)),
            pl.BlockSpec((q, _K), lambda i: (0, 0)),
        ],
        out_shape=[
            jax.ShapeDtypeStruct((q, _K), jnp.float32),
            jax.ShapeDtypeStruct((q, _K), jnp.int32),
        ],
        scratch_shapes=[
            pltpu.VMEM((q, d), jnp.float32),
            pltpu.VMEM((q, _K), jnp.float32),
            pltpu.VMEM((q, _K), jnp.float32),
        ],
        compiler_params=pltpu.CompilerParams(
            dimension_semantics=("arbitrary",)),
    )(mask, x, k3)


def _gather_call(values, idx2d, wbc, q):
    n, d = values.shape
    qpw = q // _NW
    rows = qpw * _K
    irows = rows // 128

    @functools.partial(
        pl.kernel,
        out_type=jax.ShapeDtypeStruct((q, d), jnp.float32),
        mesh=plsc.VectorSubcoreMesh(core_axis_name="c", subcore_axis_name="s"),
        scratch_types=[
            pltpu.VMEM((irows, 128), jnp.int32),
            pltpu.VMEM((rows, _LANES), jnp.float32),
            pltpu.VMEM((rows, d), jnp.float32),
            pltpu.VMEM((qpw, d), jnp.float32),
            pltpu.SemaphoreType.DMA,
        ],
    )
    def _gather(values_hbm, idx_hbm, w_hbm, out_hbm,
                idx_v, w_v, rows_v, out_v, sem):
        wid = lax.axis_index("s") * 2 + lax.axis_index("c")
        pltpu.sync_copy(idx_hbm.at[pl.ds(wid * irows, irows)], idx_v)
        pltpu.sync_copy(w_hbm.at[pl.ds(wid * rows, rows)], w_v)
        cps = [
            pltpu.async_copy(values_hbm.at[idx_v.at[r]],
                             rows_v.at[pl.ds(r * 128, 128)], sem)
            for r in range(irows)
        ]
        for cp in cps:
            cp.wait()

        def qbody(qq, carry):
            rbase = qq * _K
            wb = [w_v[rbase + j, :] for j in range(_K)]
            for c in range(d // _LANES):
                sl = pl.ds(c * _LANES, _LANES)
                acc = wb[0] * rows_v[rbase, sl]
                for j in range(1, _K):
                    acc = acc + wb[j] * rows_v[rbase + j, sl]
                out_v[qq, sl] = acc
            return carry

        lax.fori_loop(0, qpw, qbody, 0)
        pltpu.sync_copy(out_v, out_hbm.at[pl.ds(wid * qpw, qpw)])

    return _gather(values, idx2d, wbc)


def kernel(x, keys, values, top_k):
    q, d = x.shape
    n = keys.shape[0]
    mask = (jnp.arange(_K) < jnp.minimum(top_k, n))
    mask = mask.astype(jnp.float32).reshape(1, _K)
    w, ti = _topk_call(mask, x.astype(jnp.float32), keys.astype(jnp.float32))
    idx2d = ti.reshape(-1, 128)
    wbc = jnp.broadcast_to(w.reshape(-1, 1), (q * _K, _LANES))
    out = _gather_call(values.astype(jnp.float32), idx2d, wbc, q)
    return out.astype(x.dtype)


# speedup vs baseline: 1.0129x; 1.0129x over previous
import functools

import jax
import jax.numpy as jnp
from jax import lax
from jax.experimental import pallas as pl
from jax.experimental.pallas import tpu as pltpu
from jax.experimental.pallas import tpu_sc as plsc

_K = 8
_P = 4
_G = 10
_S = 125
_BN = _P * _G * _S
_NW = 32
_LANES = 16


def _topk_body(mask_ref, x_ref, k_ref, w_ref, i_ref, xn_ref, rv_ref, ri_ref):
    q, _ = xn_ref.shape
    blk = pl.program_id(0)
    nblk = pl.num_programs(0)

    @pl.when(blk == 0)
    def _init():
        xx = x_ref[...]
        nrm = jnp.sqrt(jnp.sum(xx * xx, axis=1, keepdims=True))
        xn_ref[...] = xx / jnp.maximum(nrm, 1e-12)
        rv_ref[...] = jnp.full(rv_ref.shape, -jnp.inf, jnp.float32)
        ri_ref[...] = jnp.zeros(ri_ref.shape, jnp.float32)

    xn = xn_ref[...]
    cols = lax.broadcasted_iota(jnp.int32, (q, _S), 1).astype(jnp.float32)
    sss, pms, segs = [], [], []
    for p in range(_P):
        ssp = []
        for i in range(_G):
            kb = k_ref[p * _G + i]
            kn = jnp.sqrt(jnp.sum(kb * kb, axis=1, keepdims=True))
            kbn = kb / jnp.maximum(kn, 1e-12)
            ssp.append(lax.dot_general(xn, kbn, (((1,), (1,)), ((), ())),
                                       preferred_element_type=jnp.float32))
        sss.append(ssp)
        pm = ssp[0]
        for st in ssp[1:]:
            pm = jnp.maximum(pm, st)
        pms.append(pm)
        segs.append([])

    for _ in range(_K):
        for p in range(_P):
            m = jnp.max(pms[p], axis=1, keepdims=True)
            eq = pms[p] == m
            j = jnp.min(jnp.where(eq, cols, jnp.inf), axis=1, keepdims=True)
            pms[p] = jnp.where(eq, -jnp.inf, pms[p])
            segs[p].append(j)
    segfs = [jnp.concatenate(s, axis=1) for s in segs]

    base = blk * _BN
    cvals, reps = [], []
    for p in range(_P):
        sidx = segfs[p].astype(jnp.int32)
        cvals += [jnp.take_along_axis(s, sidx, axis=1) for s in sss[p]]
        reps += [segfs[p]] * _G
    pool = jnp.concatenate(cvals + [rv_ref[...]], axis=1)
    npool = _P * _G * _K + _K
    lane = lax.broadcasted_iota(jnp.int32, (q, npool), 1)
    offs = jnp.where(lane < _P * _G * _K, (lane // _K) * _S + base, 0)
    segrep = jnp.concatenate(reps + [ri_ref[...]], axis=1)
    poolidx = segrep + offs.astype(jnp.float32)
    ms, gs = [], []
    for _ in range(_K):
        m = jnp.max(pool, axis=1, keepdims=True)
        eq = pool == m
        g = jnp.min(jnp.where(eq, poolidx, jnp.inf), axis=1, keepdims=True)
        pool = jnp.where(eq, -jnp.inf, pool)
        ms.append(m)
        gs.append(g)
    rv_ref[...] = jnp.concatenate(ms, axis=1)
    ri_ref[...] = jnp.concatenate(gs, axis=1)

    @pl.when(blk == nblk - 1)
    def _fin():
        rv = rv_ref[...]
        valid = mask_ref[...] > 0.0
        mv = jnp.where(valid, rv, -jnp.inf)
        e = jnp.exp(mv - mv[:, :1])
        w_ref[...] = e / jnp.sum(e, axis=1, keepdims=True)
        i_ref[...] = ri_ref[...].astype(jnp.int32)


def _topk_call(mask, x, keys):
    q, d = x.shape
    n = keys.shape[0]
    nblk = n // _BN
    k3 = keys.reshape(nblk * _P * _G, _S, d)
    return pl.pallas_call(
        _topk_body,
        grid=(nblk,),
        in_specs=[
            pl.BlockSpec((1, _K), lambda i: (0, 0)),
            pl.BlockSpec((q, d), lambda i: (0, 0)),
            pl.BlockSpec((_P * _G, _S, d), lambda i: (i, 0, 0)),
        ],
        out_specs=[
            pl.BlockSpec((q, _K), lambda i: (0, 0)),
            pl.BlockSpec((q, _K), lambda i: (0, 0)),
        ],
        out_shape=[
            jax.ShapeDtypeStruct((q, _K), jnp.float32),
            jax.ShapeDtypeStruct((q, _K), jnp.int32),
        ],
        scratch_shapes=[
            pltpu.VMEM((q, d), jnp.float32),
            pltpu.VMEM((q, _K), jnp.float32),
            pltpu.VMEM((q, _K), jnp.float32),
        ],
        compiler_params=pltpu.CompilerParams(
            dimension_semantics=("arbitrary",)),
    )(mask, x, k3)


def _gather_call(values, idx2d, wbc, q):
    n, d = values.shape
    qpw = q // _NW
    rows = qpw * _K
    irows = rows // 128

    @functools.partial(
        pl.kernel,
        out_type=jax.ShapeDtypeStruct((q, d), jnp.float32),
        mesh=plsc.VectorSubcoreMesh(core_axis_name="c", subcore_axis_name="s"),
        scratch_types=[
            pltpu.VMEM((irows, 128), jnp.int32),
            pltpu.VMEM((rows, _LANES), jnp.float32),
            pltpu.VMEM((rows, d), jnp.float32),
            pltpu.VMEM((qpw, d), jnp.float32),
            pltpu.SemaphoreType.DMA,
        ],
    )
    def _gather(values_hbm, idx_hbm, w_hbm, out_hbm,
                idx_v, w_v, rows_v, out_v, sem):
        wid = lax.axis_index("s") * 2 + lax.axis_index("c")
        pltpu.sync_copy(idx_hbm.at[pl.ds(wid * irows, irows)], idx_v)
        pltpu.sync_copy(w_hbm.at[pl.ds(wid * rows, rows)], w_v)
        cps = [
            pltpu.async_copy(values_hbm.at[idx_v.at[r]],
                             rows_v.at[pl.ds(r * 128, 128)], sem)
            for r in range(irows)
        ]
        for cp in cps:
            cp.wait()

        def qbody(qq, carry):
            rbase = qq * _K
            wb = [w_v[rbase + j, :] for j in range(_K)]
            for c in range(d // _LANES):
                sl = pl.ds(c * _LANES, _LANES)
                acc = wb[0] * rows_v[rbase, sl]
                for j in range(1, _K):
                    acc = acc + wb[j] * rows_v[rbase + j, sl]
                out_v[qq, sl] = acc
            return carry

        lax.fori_loop(0, qpw, qbody, 0)
        pltpu.sync_copy(out_v, out_hbm.at[pl.ds(wid * qpw, qpw)])

    return _gather(values, idx2d, wbc)


def kernel(x, keys, values, top_k):
    q, d = x.shape
    n = keys.shape[0]
    mask = (jnp.arange(_K) < jnp.minimum(top_k, n))
    mask = mask.astype(jnp.float32).reshape(1, _K)
    w, ti = _topk_call(mask, x.astype(jnp.float32), keys.astype(jnp.float32))
    idx2d = ti.reshape(-1, 128)
    wbc = jnp.broadcast_to(w.reshape(-1, 1), (q * _K, _LANES))
    out = _gather_call(values.astype(jnp.float32), idx2d, wbc, q)
    return out.astype(x.dtype)
